# Initial kernel scaffold; baseline (speedup 1.0000x reference)
import jax
import jax.numpy as jnp
from jax.experimental import pallas as pl


def kernel(x, positions, pos_emb):
    raise NotImplementedError("write your pallas kernel here")


# speedup vs baseline: 1.1657x; 1.1657x over previous
import functools

import jax
import jax.numpy as jnp
from jax import lax
from jax.experimental import pallas as pl
from jax.experimental.pallas import tpu as pltpu
from jax.experimental.pallas import tpu_sc as plsc


def _build(N, D, V, rows_per_worker, K):
    chunks = rows_per_worker // K
    mesh = plsc.VectorSubcoreMesh(core_axis_name="c", subcore_axis_name="s")
    nc = mesh.num_cores

    def body(x_hbm, idx_hbm, tab_hbm, out_hbm, idx_v, buf, rbuf, sem):
        wid = lax.axis_index("s") * nc + lax.axis_index("c")
        base = wid * rows_per_worker

        def chunk(i, _):
            start = base + i * K
            pltpu.sync_copy(idx_hbm.at[pl.ds(start, K)], idx_v)
            gather = pltpu.async_copy(tab_hbm.at[idx_v], rbuf, sem)
            pltpu.sync_copy(x_hbm.at[pl.ds(start, K)], buf)
            gather.wait()

            @plsc.parallel_loop(0, K)
            def add_row(r):
                for j in range(D // 16):
                    sl = pl.ds(j * 16, 16)
                    plsc.addupdate(buf.at[r, sl], rbuf[r, sl])

            pltpu.sync_copy(buf, out_hbm.at[pl.ds(start, K)])
            return 0

        lax.fori_loop(0, chunks, chunk, 0)

    return pl.kernel(
        body,
        out_type=jax.ShapeDtypeStruct((N, D), jnp.float32),
        mesh=mesh,
        scratch_types=[
            pltpu.VMEM((K,), jnp.int32),
            pltpu.VMEM((K, D), jnp.float32),
            pltpu.VMEM((K, D), jnp.float32),
            pltpu.SemaphoreType.DMA,
        ],
    )


@jax.jit
def kernel(x, positions, pos_emb):
    B, S, D = x.shape
    V = pos_emb.shape[0]
    N = B * S
    nw = 32
    rows_per_worker = N // nw
    fn = _build(N, D, V, rows_per_worker, K=32)
    out = fn(x.reshape(N, D), positions.reshape(N), pos_emb)
    return out.reshape(B, S, D)


# speedup vs baseline: 1.4900x; 1.2783x over previous
import functools

import jax
import jax.numpy as jnp
from jax import lax
from jax.experimental import pallas as pl
from jax.experimental.pallas import tpu as pltpu
from jax.experimental.pallas import tpu_sc as plsc


def _build(N, D, rows_per_worker, K):
    chunks = rows_per_worker // K
    npairs = chunks // 2
    mesh = plsc.VectorSubcoreMesh(core_axis_name="c", subcore_axis_name="s")
    nc = mesh.num_cores

    def body(x_hbm, idx_hbm, tab_hbm, out_hbm,
             idx_all, buf0, buf1, rbuf0, rbuf1,
             sg0, sg1, sx0, sx1, so0, so1):
        wid = lax.axis_index("s") * nc + lax.axis_index("c")
        base = wid * rows_per_worker

        pltpu.sync_copy(idx_hbm.at[pl.ds(base, rows_per_worker)], idx_all)

        def idx_sl(i):
            return idx_all.at[pl.ds(i * K, K)]

        def start_gather(i, rbuf, sem):
            pltpu.async_copy(tab_hbm.at[idx_sl(i)], rbuf, sem)

        def start_x(i, buf, sem):
            pltpu.async_copy(x_hbm.at[pl.ds(base + i * K, K)], buf, sem)

        def wait_gather(rbuf, sem):
            pltpu.make_async_copy(x_hbm.at[pl.ds(0, K)], rbuf, sem).wait()

        def wait_x(buf, sem):
            pltpu.make_async_copy(x_hbm.at[pl.ds(0, K)], buf, sem).wait()

        def start_out(i, buf, sem):
            pltpu.async_copy(buf, out_hbm.at[pl.ds(base + i * K, K)], sem)

        def wait_out(i, buf, sem):
            pltpu.make_async_copy(buf, out_hbm.at[pl.ds(base + i * K, K)], sem).wait()

        def add_chunk(buf, rbuf):
            @plsc.parallel_loop(0, K)
            def add_row(r):
                for j in range(D // 16):
                    sl = pl.ds(j * 16, 16)
                    plsc.addupdate(buf.at[r, sl], rbuf[r, sl])

        start_gather(0, rbuf0, sg0)
        start_x(0, buf0, sx0)
        start_gather(1, rbuf1, sg1)
        start_x(1, buf1, sx1)

        def pair(p, _):
            i0 = 2 * p
            i1 = i0 + 1
            more = p + 1 < npairs

            wait_gather(rbuf0, sg0)
            wait_x(buf0, sx0)
            add_chunk(buf0, rbuf0)
            start_out(i0, buf0, so0)

            @pl.when(more)
            def _():
                start_gather(i0 + 2, rbuf0, sg0)

            wait_gather(rbuf1, sg1)
            wait_x(buf1, sx1)
            add_chunk(buf1, rbuf1)
            start_out(i1, buf1, so1)

            @pl.when(more)
            def _():
                start_gather(i1 + 2, rbuf1, sg1)

            wait_out(i0, buf0, so0)

            @pl.when(more)
            def _():
                start_x(i0 + 2, buf0, sx0)

            wait_out(i1, buf1, so1)

            @pl.when(more)
            def _():
                start_x(i1 + 2, buf1, sx1)

            return 0

        lax.fori_loop(0, npairs, pair, 0)

    return pl.kernel(
        body,
        out_type=jax.ShapeDtypeStruct((N, D), jnp.float32),
        mesh=mesh,
        scratch_types=[
            pltpu.VMEM((rows_per_worker,), jnp.int32),
            pltpu.VMEM((K, D), jnp.float32),
            pltpu.VMEM((K, D), jnp.float32),
            pltpu.VMEM((K, D), jnp.float32),
            pltpu.VMEM((K, D), jnp.float32),
            pltpu.SemaphoreType.DMA,
            pltpu.SemaphoreType.DMA,
            pltpu.SemaphoreType.DMA,
            pltpu.SemaphoreType.DMA,
            pltpu.SemaphoreType.DMA,
            pltpu.SemaphoreType.DMA,
        ],
    )


@jax.jit
def kernel(x, positions, pos_emb):
    B, S, D = x.shape
    N = B * S
    nw = 32
    rows_per_worker = N // nw
    fn = _build(N, D, rows_per_worker, K=16)
    out = fn(x.reshape(N, D), positions.reshape(N), pos_emb)
    return out.reshape(B, S, D)


# speedup vs baseline: 1.6287x; 1.0931x over previous
import functools

import jax
import jax.numpy as jnp
from jax import lax
from jax.experimental import pallas as pl
from jax.experimental.pallas import tpu as pltpu
from jax.experimental.pallas import tpu_sc as plsc


def _build(N, D, rows_per_worker, K):
    chunks = rows_per_worker // K
    nquads = chunks // 4
    mesh = plsc.VectorSubcoreMesh(core_axis_name="c", subcore_axis_name="s")
    nc = mesh.num_cores

    def body(x_hbm, idx_hbm, tab_hbm, out_hbm,
             idx_all, b0, b1, b2, b3, r0, r1,
             sg0, sg1, sx0, sx1, sx2, sx3, so0, so1, so2, so3):
        wid = lax.axis_index("s") * nc + lax.axis_index("c")
        base = wid * rows_per_worker

        bufs = (b0, b1, b2, b3)
        rbufs = (r0, r1)
        sxs = (sx0, sx1, sx2, sx3)
        sos = (so0, so1, so2, so3)

        pltpu.sync_copy(idx_hbm.at[pl.ds(base, rows_per_worker)], idx_all)

        def start_gather(j, r, sem):
            pltpu.async_copy(tab_hbm.at[idx_all.at[pl.ds(j * K, K)]], r, sem)

        def start_x(j, buf, sem):
            pltpu.async_copy(x_hbm.at[pl.ds(base + j * K, K)], buf, sem)

        def wait_into(buf, sem):
            pltpu.make_async_copy(x_hbm.at[pl.ds(0, K)], buf, sem).wait()

        def start_out(j, buf, sem):
            pltpu.async_copy(buf, out_hbm.at[pl.ds(base + j * K, K)], sem)

        def wait_out(j, buf, sem):
            pltpu.make_async_copy(buf, out_hbm.at[pl.ds(base + j * K, K)], sem).wait()

        def add_chunk(buf, rbuf):
            @plsc.parallel_loop(0, K, unroll=2)
            def add_row(r):
                for j in range(D // 16):
                    sl = pl.ds(j * 16, 16)
                    plsc.addupdate(buf.at[r, sl], rbuf[r, sl])

        start_gather(0, r0, sg0)
        start_x(0, b0, sx0)
        start_gather(1, r1, sg1)
        start_x(1, b1, sx1)
        start_x(2, b2, sx2)
        start_x(3, b3, sx3)

        def quad(q, _):
            j0 = 4 * q
            for k in range(4):
                j = j0 + k
                buf, sx, so = bufs[k], sxs[k], sos[k]
                rb, sg = rbufs[k % 2], (sg0, sg1)[k % 2]

                wait_into(rb, sg)
                wait_into(buf, sx)
                add_chunk(buf, rb)
                start_out(j, buf, so)

                @pl.when(j + 2 < chunks)
                def _():
                    start_gather(j + 2, rb, sg)

                @pl.when(j >= 2)
                def _():
                    pbuf = bufs[(k + 2) % 4]
                    wait_out(j - 2, pbuf, sos[(k + 2) % 4])

                    @pl.when(j + 2 < chunks)
                    def _():
                        start_x(j + 2, pbuf, sxs[(k + 2) % 4])

            return 0

        lax.fori_loop(0, nquads, quad, 0)

        wait_out(chunks - 2, bufs[(chunks - 2) % 4], sos[(chunks - 2) % 4])
        wait_out(chunks - 1, bufs[(chunks - 1) % 4], sos[(chunks - 1) % 4])

    return pl.kernel(
        body,
        out_type=jax.ShapeDtypeStruct((N, D), jnp.float32),
        mesh=mesh,
        scratch_types=[
            pltpu.VMEM((rows_per_worker,), jnp.int32),
            pltpu.VMEM((K, D), jnp.float32),
            pltpu.VMEM((K, D), jnp.float32),
            pltpu.VMEM((K, D), jnp.float32),
            pltpu.VMEM((K, D), jnp.float32),
            pltpu.VMEM((K, D), jnp.float32),
            pltpu.VMEM((K, D), jnp.float32),
        ] + [pltpu.SemaphoreType.DMA] * 10,
    )


@jax.jit
def kernel(x, positions, pos_emb):
    B, S, D = x.shape
    N = B * S
    nw = 32
    rows_per_worker = N // nw
    fn = _build(N, D, rows_per_worker, K=16)
    out = fn(x.reshape(N, D), positions.reshape(N), pos_emb)
    return out.reshape(B, S, D)
